# baseline (speedup 1.0000x reference)
import functools

import jax
import jax.numpy as jnp
from jax import lax
from jax.experimental import pallas as pl
from jax.experimental.pallas import tpu as pltpu
from jax.experimental.pallas import tpu_sc as plsc

N = 10000
E = 320000
D = 128
C_OUT = 64

NC = 2
NS = 16
NW = NC * NS
EPT = E // NW
K = 80
CH = EPT // K
ZB = 624
NB = 3
F = 1
MAIN = (CH // NB) * NB


def _segment_sum_sc(x, src3, dst3, zrows):
    mesh = plsc.VectorSubcoreMesh(core_axis_name="c", subcore_axis_name="s")

    @functools.partial(
        pl.kernel,
        out_type=jax.ShapeDtypeStruct((NC, N, D), jnp.float32),
        mesh=mesh,
        scratch_types=[
            pltpu.VMEM((CH, K), jnp.int32),
            pltpu.VMEM((CH, K), jnp.int32),
            pltpu.VMEM((NB, K, D), jnp.float32),
            pltpu.VMEM_SHARED((N, D), jnp.float32),
            [pltpu.SemaphoreType.DMA] * NB,
            [pltpu.SemaphoreType.DMA] * NB,
        ],
        compiler_params=pltpu.CompilerParams(use_tc_tiling_on_sc=False),
    )
    def seg_kernel(x_hbm, src_hbm, dst_hbm, z_hbm, out_hbm,
                   src_v, dst_v, rows, acc_sh, g_sems, s_sems):
        cid = lax.axis_index("c")
        sid = lax.axis_index("s")
        tile = cid * NS + sid

        pltpu.sync_copy(src_hbm.at[tile], src_v)
        pltpu.sync_copy(dst_hbm.at[tile], dst_v)
        for b in range(F):
            pltpu.async_copy(x_hbm.at[src_v.at[b]], rows.at[b], g_sems[b])

        pltpu.sync_copy(z_hbm, acc_sh.at[pl.ds(sid * ZB, ZB)])

        @pl.when(sid == NS - 1)
        def _zero_tail():
            pltpu.sync_copy(z_hbm.at[pl.ds(0, N - NS * ZB)],
                            acc_sh.at[pl.ds(NS * ZB, N - NS * ZB)])

        plsc.subcore_barrier()

        def step(j, b):
            pltpu.make_async_copy(x_hbm.at[src_v.at[j]],
                                  rows.at[b], g_sems[b]).wait()
            pltpu.async_copy(rows.at[b], acc_sh.at[dst_v.at[j]],
                             s_sems[b], add=True)
            bb = (b + F) % NB

            @pl.when(jnp.logical_and(jnp.asarray(j + F < CH),
                                     jnp.asarray(j + F >= NB)))
            def _wait_prev_scatter():
                pltpu.make_async_copy(rows.at[bb],
                                      acc_sh.at[dst_v.at[j]],
                                      s_sems[bb]).wait()

            @pl.when(jnp.asarray(j + F < CH))
            def _start_gather():
                pltpu.async_copy(x_hbm.at[src_v.at[j + F]],
                                 rows.at[bb], g_sems[bb])

        def block(base, carry):
            for b in range(NB):
                step(base + b, b)
            return carry

        lax.fori_loop(0, MAIN // NB, lambda m, c: block(m * NB, c), None)
        for jt in range(MAIN, CH):
            step(jt, jt % NB)
        for j2 in range(CH - NB, CH):
            b2 = j2 % NB
            pltpu.make_async_copy(rows.at[b2], acc_sh.at[dst_v.at[0]],
                                  s_sems[b2]).wait()

        plsc.subcore_barrier()

        pltpu.sync_copy(acc_sh.at[pl.ds(sid * ZB, ZB)],
                        out_hbm.at[cid, pl.ds(sid * ZB, ZB)])

        @pl.when(sid == NS - 1)
        def _out_tail():
            pltpu.sync_copy(acc_sh.at[pl.ds(NS * ZB, N - NS * ZB)],
                            out_hbm.at[cid, pl.ds(NS * ZB, N - NS * ZB)])

    return seg_kernel(x, src3, dst3, zrows)


def _layer_tc(p, w):
    RB = 1000

    def body(p_ref, w_ref, o_ref):
        a = p_ref[0] + p_ref[1]
        h = jnp.dot(a, w_ref[...], preferred_element_type=jnp.float32)
        o_ref[...] = jnp.maximum(h, 0.0)

    return pl.pallas_call(
        body,
        grid=(N // RB,),
        in_specs=[
            pl.BlockSpec((NC, RB, D), lambda i: (0, i, 0)),
            pl.BlockSpec((D, D), lambda i: (0, 0)),
        ],
        out_specs=pl.BlockSpec((RB, D), lambda i: (i, 0)),
        out_shape=jax.ShapeDtypeStruct((N, D), jnp.float32),
    )(p, w)


def _final_tc(p, w, cls):
    RB = 1000

    def body(p_ref, w_ref, c_ref, o_ref):
        a = p_ref[0] + p_ref[1]
        h = jnp.dot(a, w_ref[...], preferred_element_type=jnp.float32)
        h = jnp.maximum(h, 0.0)
        z = jnp.dot(h, c_ref[...], preferred_element_type=jnp.float32)
        o_ref[...] = jnp.where(z > 0, 1.0, 0.0)

    return pl.pallas_call(
        body,
        grid=(N // RB,),
        in_specs=[
            pl.BlockSpec((NC, RB, D), lambda i: (0, i, 0)),
            pl.BlockSpec((D, D), lambda i: (0, 0)),
            pl.BlockSpec((D, C_OUT), lambda i: (0, 0)),
        ],
        out_specs=pl.BlockSpec((RB, C_OUT), lambda i: (i, 0)),
        out_shape=jax.ShapeDtypeStruct((N, C_OUT), jnp.float32),
    )(p, w, cls)


def kernel(x, edge_index, weight_list, classifier):
    dst = edge_index[0].astype(jnp.int32).reshape(NW, CH, K)
    src = edge_index[1].astype(jnp.int32).reshape(NW, CH, K)
    zrows = jnp.zeros((ZB, D), jnp.float32)

    p = _segment_sum_sc(x, src, dst, zrows)
    h = _layer_tc(p, weight_list[0])
    p = _segment_sum_sc(h, src, dst, zrows)
    return _final_tc(p, weight_list[1], classifier)


# speedup vs baseline: 1.4319x; 1.4319x over previous
import functools

import jax
import jax.numpy as jnp
from jax import lax
from jax.experimental import pallas as pl
from jax.experimental.pallas import tpu as pltpu
from jax.experimental.pallas import tpu_sc as plsc

N = 10000
E = 320000
D = 128
C_OUT = 64

NC = 2
NS = 16
NW = NC * NS
EPT = E // NW
K = 80
CH = EPT // K
ZB = 624
NB = 3
F = 2
MAIN = (CH // NB) * NB


def _segment_sum_sc(x, src3, dst3, zrows):
    mesh = plsc.VectorSubcoreMesh(core_axis_name="c", subcore_axis_name="s")

    @functools.partial(
        pl.kernel,
        out_type=jax.ShapeDtypeStruct((NC, N, D), jnp.float32),
        mesh=mesh,
        scratch_types=[
            pltpu.VMEM((CH, K), jnp.int32),
            pltpu.VMEM((CH, K), jnp.int32),
            pltpu.VMEM((NB, K, D), jnp.float32),
            pltpu.VMEM_SHARED((N, D), jnp.float32),
            [pltpu.SemaphoreType.DMA] * NB,
            [pltpu.SemaphoreType.DMA] * NB,
        ],
        compiler_params=pltpu.CompilerParams(use_tc_tiling_on_sc=False),
    )
    def seg_kernel(x_hbm, src_hbm, dst_hbm, z_hbm, out_hbm,
                   src_v, dst_v, rows, acc_sh, g_sems, s_sems):
        cid = lax.axis_index("c")
        sid = lax.axis_index("s")
        tile = cid * NS + sid

        pltpu.sync_copy(src_hbm.at[tile], src_v)
        pltpu.sync_copy(dst_hbm.at[tile], dst_v)
        for b in range(F):
            pltpu.async_copy(x_hbm.at[src_v.at[b]], rows.at[b], g_sems[b])

        pltpu.sync_copy(z_hbm, acc_sh.at[pl.ds(sid * ZB, ZB)])

        @pl.when(sid == NS - 1)
        def _zero_tail():
            pltpu.sync_copy(z_hbm.at[pl.ds(0, N - NS * ZB)],
                            acc_sh.at[pl.ds(NS * ZB, N - NS * ZB)])

        plsc.subcore_barrier()

        def step(j, b):
            pltpu.make_async_copy(x_hbm.at[src_v.at[j]],
                                  rows.at[b], g_sems[b]).wait()
            pltpu.async_copy(rows.at[b], acc_sh.at[dst_v.at[j]],
                             s_sems[b], add=True)
            bb = (b + F) % NB

            @pl.when(jnp.logical_and(jnp.asarray(j + F < CH),
                                     jnp.asarray(j + F >= NB)))
            def _wait_prev_scatter():
                pltpu.make_async_copy(rows.at[bb],
                                      acc_sh.at[dst_v.at[j]],
                                      s_sems[bb]).wait()

            @pl.when(jnp.asarray(j + F < CH))
            def _start_gather():
                pltpu.async_copy(x_hbm.at[src_v.at[j + F]],
                                 rows.at[bb], g_sems[bb])

        def block(base, carry):
            for b in range(NB):
                step(base + b, b)
            return carry

        lax.fori_loop(0, MAIN // NB, lambda m, c: block(m * NB, c), None)
        for jt in range(MAIN, CH):
            step(jt, jt % NB)
        for j2 in range(CH - NB, CH):
            b2 = j2 % NB
            pltpu.make_async_copy(rows.at[b2], acc_sh.at[dst_v.at[0]],
                                  s_sems[b2]).wait()

        plsc.subcore_barrier()

        pltpu.sync_copy(acc_sh.at[pl.ds(sid * ZB, ZB)],
                        out_hbm.at[cid, pl.ds(sid * ZB, ZB)])

        @pl.when(sid == NS - 1)
        def _out_tail():
            pltpu.sync_copy(acc_sh.at[pl.ds(NS * ZB, N - NS * ZB)],
                            out_hbm.at[cid, pl.ds(NS * ZB, N - NS * ZB)])

    return seg_kernel(x, src3, dst3, zrows)


def _layer_tc(p, w):
    RB = 1000

    def body(p_ref, w_ref, o_ref):
        a = p_ref[0] + p_ref[1]
        h = jnp.dot(a, w_ref[...], preferred_element_type=jnp.float32)
        o_ref[...] = jnp.maximum(h, 0.0)

    return pl.pallas_call(
        body,
        grid=(N // RB,),
        in_specs=[
            pl.BlockSpec((NC, RB, D), lambda i: (0, i, 0)),
            pl.BlockSpec((D, D), lambda i: (0, 0)),
        ],
        out_specs=pl.BlockSpec((RB, D), lambda i: (i, 0)),
        out_shape=jax.ShapeDtypeStruct((N, D), jnp.float32),
    )(p, w)


def _final_tc(p, w, cls):
    RB = 1000

    def body(p_ref, w_ref, c_ref, o_ref):
        a = p_ref[0] + p_ref[1]
        h = jnp.dot(a, w_ref[...], preferred_element_type=jnp.float32)
        h = jnp.maximum(h, 0.0)
        z = jnp.dot(h, c_ref[...], preferred_element_type=jnp.float32)
        o_ref[...] = jnp.where(z > 0, 1.0, 0.0)

    return pl.pallas_call(
        body,
        grid=(N // RB,),
        in_specs=[
            pl.BlockSpec((NC, RB, D), lambda i: (0, i, 0)),
            pl.BlockSpec((D, D), lambda i: (0, 0)),
            pl.BlockSpec((D, C_OUT), lambda i: (0, 0)),
        ],
        out_specs=pl.BlockSpec((RB, C_OUT), lambda i: (i, 0)),
        out_shape=jax.ShapeDtypeStruct((N, C_OUT), jnp.float32),
    )(p, w, cls)


def kernel(x, edge_index, weight_list, classifier):
    dst = edge_index[0].astype(jnp.int32).reshape(NW, CH, K)
    src = edge_index[1].astype(jnp.int32).reshape(NW, CH, K)
    zrows = jnp.zeros((ZB, D), jnp.float32)

    p = _segment_sum_sc(x, src, dst, zrows)
    h = _layer_tc(p, weight_list[0])
    p = _segment_sum_sc(h, src, dst, zrows)
    return _final_tc(p, weight_list[1], classifier)
